# baseline (speedup 1.0000x reference)
import functools

import jax
import jax.numpy as jnp
from jax import lax
from jax.experimental import pallas as pl
from jax.experimental.pallas import tpu as pltpu

_N = 4194304
_COLS = 1024
_ROWS = _N // _COLS
_CHROWS = 2048
_NCH = _ROWS // _CHROWS
_DEPTH = 2


def _tc_body(y_hbm, s_hbm, oy_ref, oys_ref, os_ref,
             ybuf, sbuf, sems_y, sems_s):
    def start(k):
        slot = k % _DEPTH
        cy = pltpu.make_async_copy(
            y_hbm.at[pl.ds(k * _CHROWS, _CHROWS), :], ybuf.at[slot],
            sems_y.at[slot])
        cs = pltpu.make_async_copy(
            s_hbm.at[pl.ds(k * _CHROWS, _CHROWS), :], sbuf.at[slot],
            sems_s.at[slot])
        cy.start(priority=0)
        cs.start(priority=1)
        return cy, cs

    pending = [start(k) for k in range(_DEPTH)]

    zero = jnp.zeros((8, _COLS), jnp.float32)
    acc = (zero, zero, zero)
    for k in range(_NCH):
        slot = k % _DEPTH
        cy, cs = pending[slot]
        cy.wait()
        cs.wait()

        def body(i, a, _slot=slot):
            ay, ays, asf = a
            yv = ybuf[_slot, pl.ds(i * 8, 8), :]
            sv = sbuf[_slot, pl.ds(i * 8, 8), :]
            ay = ay + yv
            ays = ays + jnp.where(sv == 1, yv, jnp.float32(0.0))
            asf = asf + sv.astype(jnp.float32)
            return (ay, ays, asf)

        acc = lax.fori_loop(0, _CHROWS // 8, body, acc, unroll=2)
        if k + _DEPTH < _NCH:
            pending[slot] = start(k + _DEPTH)

    oy_ref[0, 0] = jnp.sum(acc[0])
    oys_ref[0, 0] = jnp.sum(acc[1])
    os_ref[0, 0] = jnp.sum(acc[2])


_tc_reduce = pl.pallas_call(
    _tc_body,
    in_specs=[
        pl.BlockSpec(memory_space=pl.ANY),
        pl.BlockSpec(memory_space=pl.ANY),
    ],
    out_specs=[
        pl.BlockSpec(memory_space=pltpu.SMEM),
        pl.BlockSpec(memory_space=pltpu.SMEM),
        pl.BlockSpec(memory_space=pltpu.SMEM),
    ],
    out_shape=[
        jax.ShapeDtypeStruct((1, 1), jnp.float32),
        jax.ShapeDtypeStruct((1, 1), jnp.float32),
        jax.ShapeDtypeStruct((1, 1), jnp.float32),
    ],
    scratch_shapes=[
        pltpu.VMEM((_DEPTH, _CHROWS, _COLS), jnp.float32),
        pltpu.VMEM((_DEPTH, _CHROWS, _COLS), jnp.int32),
        pltpu.SemaphoreType.DMA((_DEPTH,)),
        pltpu.SemaphoreType.DMA((_DEPTH,)),
    ],
)


def kernel(y_pred, s):
    y2 = y_pred.reshape(_ROWS, _COLS)
    s2 = s.reshape(_ROWS, _COLS)
    sy, sys_, cnt1 = _tc_reduce(y2, s2)
    sum_y = sy[0, 0]
    sum_ys = sys_[0, 0]
    c1 = cnt1[0, 0]
    c0 = jnp.float32(_N) - c1
    mean1 = sum_ys / c1
    mean0 = (sum_y - sum_ys) / c0
    return jnp.abs(mean0 - mean1)


# speedup vs baseline: 2.5403x; 2.5403x over previous
import functools

import jax
import jax.numpy as jnp
from jax import lax
from jax.experimental import pallas as pl
from jax.experimental.pallas import tpu as pltpu

_N = 4194304
_COLS = 128
_ROWS = _N // _COLS
_CHROWS = 16384
_NCH = _ROWS // _CHROWS
_DEPTH = 2


def _tc_body(y_hbm, s_hbm, oy_ref, oys_ref, os_ref,
             ybuf, sbuf, sems_y, sems_s):
    def start(k):
        slot = k % _DEPTH
        cy = pltpu.make_async_copy(
            y_hbm.at[pl.ds(k * _CHROWS, _CHROWS), :], ybuf.at[slot],
            sems_y.at[slot])
        cs = pltpu.make_async_copy(
            s_hbm.at[pl.ds(k * _CHROWS, _CHROWS), :], sbuf.at[slot],
            sems_s.at[slot])
        cy.start(priority=0)
        cs.start(priority=1)
        return cy, cs

    pending = [start(k) for k in range(_DEPTH)]

    zero = jnp.zeros((8, _COLS), jnp.float32)
    acc = (zero, zero, zero)
    for k in range(_NCH):
        slot = k % _DEPTH
        cy, cs = pending[slot]
        cy.wait()
        cs.wait()

        def body(i, a, _slot=slot):
            ay, ays, asf = a
            yv = ybuf[_slot, pl.ds(i * 8, 8), :]
            sv = sbuf[_slot, pl.ds(i * 8, 8), :]
            ay = ay + yv
            ays = ays + jnp.where(sv == 1, yv, jnp.float32(0.0))
            asf = asf + sv.astype(jnp.float32)
            return (ay, ays, asf)

        acc = lax.fori_loop(0, _CHROWS // 8, body, acc, unroll=4)
        if k + _DEPTH < _NCH:
            pending[slot] = start(k + _DEPTH)

    oy_ref[0, 0] = jnp.sum(acc[0])
    oys_ref[0, 0] = jnp.sum(acc[1])
    os_ref[0, 0] = jnp.sum(acc[2])


_tc_reduce = pl.pallas_call(
    _tc_body,
    in_specs=[
        pl.BlockSpec(memory_space=pl.ANY),
        pl.BlockSpec(memory_space=pl.ANY),
    ],
    out_specs=[
        pl.BlockSpec(memory_space=pltpu.SMEM),
        pl.BlockSpec(memory_space=pltpu.SMEM),
        pl.BlockSpec(memory_space=pltpu.SMEM),
    ],
    out_shape=[
        jax.ShapeDtypeStruct((1, 1), jnp.float32),
        jax.ShapeDtypeStruct((1, 1), jnp.float32),
        jax.ShapeDtypeStruct((1, 1), jnp.float32),
    ],
    scratch_shapes=[
        pltpu.VMEM((_DEPTH, _CHROWS, _COLS), jnp.float32),
        pltpu.VMEM((_DEPTH, _CHROWS, _COLS), jnp.int32),
        pltpu.SemaphoreType.DMA((_DEPTH,)),
        pltpu.SemaphoreType.DMA((_DEPTH,)),
    ],
    compiler_params=pltpu.CompilerParams(
        vmem_limit_bytes=56 * 1024 * 1024,
    ),
)


def kernel(y_pred, s):
    y2 = y_pred.reshape(_ROWS, _COLS)
    s2 = s.reshape(_ROWS, _COLS)
    sy, sys_, cnt1 = _tc_reduce(y2, s2)
    sum_y = sy[0, 0]
    sum_ys = sys_[0, 0]
    c1 = cnt1[0, 0]
    c0 = jnp.float32(_N) - c1
    mean1 = sum_ys / c1
    mean0 = (sum_y - sum_ys) / c0
    return jnp.abs(mean0 - mean1)


# speedup vs baseline: 3.2831x; 1.2924x over previous
import functools

import jax
import jax.numpy as jnp
from jax import lax
from jax.experimental import pallas as pl
from jax.experimental.pallas import tpu as pltpu

_N = 4194304
_COLS = 128
_ROWS = _N // _COLS
_CHROWS = 8192
_NCH = _ROWS // _CHROWS
_DEPTH = 4


def _tc_body(y_hbm, s_hbm, out_ref,
             ybuf, sbuf, sems_y, sems_s):
    def start(k):
        slot = k % _DEPTH
        cy = pltpu.make_async_copy(
            y_hbm.at[pl.ds(k * _CHROWS, _CHROWS), :], ybuf.at[slot],
            sems_y.at[slot])
        cs = pltpu.make_async_copy(
            s_hbm.at[pl.ds(k * _CHROWS, _CHROWS), :], sbuf.at[slot],
            sems_s.at[slot])
        cy.start(priority=k % 2)
        cs.start(priority=(k + 1) % 2)
        return cy, cs

    pending = [start(k) for k in range(_DEPTH)]

    zero = jnp.zeros((8, _COLS), jnp.float32)
    acc = (zero, zero, zero)
    for k in range(_NCH):
        slot = k % _DEPTH
        cy, cs = pending[slot]
        cy.wait()
        cs.wait()

        def body(i, a, _slot=slot):
            ay, ays, asf = a
            yv = ybuf[_slot, pl.ds(i * 8, 8), :]
            sv = sbuf[_slot, pl.ds(i * 8, 8), :]
            ay = ay + yv
            ays = ays + jnp.where(sv == 1, yv, jnp.float32(0.0))
            asf = asf + sv.astype(jnp.float32)
            return (ay, ays, asf)

        acc = lax.fori_loop(0, _CHROWS // 8, body, acc, unroll=4)
        if k + _DEPTH < _NCH:
            pending[slot] = start(k + _DEPTH)

    sum_y = jnp.sum(acc[0])
    sum_ys = jnp.sum(acc[1])
    c1 = jnp.sum(acc[2])
    c0 = jnp.float32(_N) - c1
    mean1 = sum_ys / c1
    mean0 = (sum_y - sum_ys) / c0
    out_ref[0, 0] = jnp.abs(mean0 - mean1)


_tc_reduce = pl.pallas_call(
    _tc_body,
    in_specs=[
        pl.BlockSpec(memory_space=pl.ANY),
        pl.BlockSpec(memory_space=pl.ANY),
    ],
    out_specs=pl.BlockSpec(memory_space=pltpu.SMEM),
    out_shape=jax.ShapeDtypeStruct((1, 1), jnp.float32),
    scratch_shapes=[
        pltpu.VMEM((_DEPTH, _CHROWS, _COLS), jnp.float32),
        pltpu.VMEM((_DEPTH, _CHROWS, _COLS), jnp.int32),
        pltpu.SemaphoreType.DMA((_DEPTH,)),
        pltpu.SemaphoreType.DMA((_DEPTH,)),
    ],
    compiler_params=pltpu.CompilerParams(
        vmem_limit_bytes=56 * 1024 * 1024,
    ),
)


def kernel(y_pred, s):
    y2 = y_pred.reshape(_ROWS, _COLS)
    s2 = s.reshape(_ROWS, _COLS)
    out = _tc_reduce(y2, s2)
    return out[0, 0]


# speedup vs baseline: 3.9205x; 1.1942x over previous
import functools

import jax
import jax.numpy as jnp
from jax import lax
from jax.experimental import pallas as pl
from jax.experimental.pallas import tpu as pltpu

_N = 4194304
_COLS = 128
_ROWS = _N // _COLS
_CHROWS = 4096
_NCH = _ROWS // _CHROWS
_DEPTH = 8


def _tc_body(y_hbm, s_hbm, out_ref,
             ybuf, sbuf, sems_y, sems_s):
    def start(k):
        slot = k % _DEPTH
        cy = pltpu.make_async_copy(
            y_hbm.at[pl.ds(k * _CHROWS, _CHROWS), :], ybuf.at[slot],
            sems_y.at[slot])
        cs = pltpu.make_async_copy(
            s_hbm.at[pl.ds(k * _CHROWS, _CHROWS), :], sbuf.at[slot],
            sems_s.at[slot])
        cy.start(priority=k % 2)
        cs.start(priority=(k + 1) % 2)
        return cy, cs

    pending = [start(k) for k in range(_DEPTH)]

    zero = jnp.zeros((8, _COLS), jnp.float32)
    acc = (zero, zero, zero)
    for k in range(_NCH):
        slot = k % _DEPTH
        cy, cs = pending[slot]
        cy.wait()
        cs.wait()

        def body(i, a, _slot=slot):
            ay, ays, asf = a
            yv = ybuf[_slot, pl.ds(i * 8, 8), :]
            sv = sbuf[_slot, pl.ds(i * 8, 8), :]
            ay = ay + yv
            ays = ays + jnp.where(sv == 1, yv, jnp.float32(0.0))
            asf = asf + sv.astype(jnp.float32)
            return (ay, ays, asf)

        acc = lax.fori_loop(0, _CHROWS // 8, body, acc, unroll=8)
        if k + _DEPTH < _NCH:
            pending[slot] = start(k + _DEPTH)

    sum_y = jnp.sum(acc[0])
    sum_ys = jnp.sum(acc[1])
    c1 = jnp.sum(acc[2])
    c0 = jnp.float32(_N) - c1
    mean1 = sum_ys / c1
    mean0 = (sum_y - sum_ys) / c0
    out_ref[0, 0] = jnp.abs(mean0 - mean1)


_tc_reduce = pl.pallas_call(
    _tc_body,
    in_specs=[
        pl.BlockSpec(memory_space=pl.ANY),
        pl.BlockSpec(memory_space=pl.ANY),
    ],
    out_specs=pl.BlockSpec(memory_space=pltpu.SMEM),
    out_shape=jax.ShapeDtypeStruct((1, 1), jnp.float32),
    scratch_shapes=[
        pltpu.VMEM((_DEPTH, _CHROWS, _COLS), jnp.float32),
        pltpu.VMEM((_DEPTH, _CHROWS, _COLS), jnp.int32),
        pltpu.SemaphoreType.DMA((_DEPTH,)),
        pltpu.SemaphoreType.DMA((_DEPTH,)),
    ],
    compiler_params=pltpu.CompilerParams(
        vmem_limit_bytes=56 * 1024 * 1024,
    ),
)


def kernel(y_pred, s):
    y2 = y_pred.reshape(_ROWS, _COLS)
    s2 = s.reshape(_ROWS, _COLS)
    out = _tc_reduce(y2, s2)
    return out[0, 0]


# speedup vs baseline: 3.9468x; 1.0067x over previous
import functools

import jax
import jax.numpy as jnp
from jax import lax
from jax.experimental import pallas as pl
from jax.experimental.pallas import tpu as pltpu

_N = 4194304
_COLS = 128
_ROWS = _N // _COLS
_CHROWS = 2048
_NCH = _ROWS // _CHROWS
_DEPTH = 16


def _tc_body(y_hbm, s_hbm, out_ref,
             ybuf, sbuf, sems_y, sems_s):
    def start(k):
        slot = k % _DEPTH
        cy = pltpu.make_async_copy(
            y_hbm.at[pl.ds(k * _CHROWS, _CHROWS), :], ybuf.at[slot],
            sems_y.at[slot])
        cs = pltpu.make_async_copy(
            s_hbm.at[pl.ds(k * _CHROWS, _CHROWS), :], sbuf.at[slot],
            sems_s.at[slot])
        cy.start(priority=k % 2)
        cs.start(priority=(k + 1) % 2)
        return cy, cs

    pending = [start(k) for k in range(_DEPTH)]

    zero = jnp.zeros((8, _COLS), jnp.float32)
    acc = (zero, zero, zero)
    for k in range(_NCH):
        slot = k % _DEPTH
        cy, cs = pending[slot]
        cy.wait()
        cs.wait()

        def body(i, a, _slot=slot):
            ay, ays, asf = a
            yv = ybuf[_slot, pl.ds(i * 8, 8), :]
            sv = sbuf[_slot, pl.ds(i * 8, 8), :]
            ay = ay + yv
            ays = ays + jnp.where(sv == 1, yv, jnp.float32(0.0))
            asf = asf + sv.astype(jnp.float32)
            return (ay, ays, asf)

        acc = lax.fori_loop(0, _CHROWS // 8, body, acc, unroll=8)
        if k + _DEPTH < _NCH:
            pending[slot] = start(k + _DEPTH)

    sum_y = jnp.sum(acc[0])
    sum_ys = jnp.sum(acc[1])
    c1 = jnp.sum(acc[2])
    c0 = jnp.float32(_N) - c1
    mean1 = sum_ys / c1
    mean0 = (sum_y - sum_ys) / c0
    out_ref[0, 0] = jnp.abs(mean0 - mean1)


_tc_reduce = pl.pallas_call(
    _tc_body,
    in_specs=[
        pl.BlockSpec(memory_space=pl.ANY),
        pl.BlockSpec(memory_space=pl.ANY),
    ],
    out_specs=pl.BlockSpec(memory_space=pltpu.SMEM),
    out_shape=jax.ShapeDtypeStruct((1, 1), jnp.float32),
    scratch_shapes=[
        pltpu.VMEM((_DEPTH, _CHROWS, _COLS), jnp.float32),
        pltpu.VMEM((_DEPTH, _CHROWS, _COLS), jnp.int32),
        pltpu.SemaphoreType.DMA((_DEPTH,)),
        pltpu.SemaphoreType.DMA((_DEPTH,)),
    ],
    compiler_params=pltpu.CompilerParams(
        vmem_limit_bytes=56 * 1024 * 1024,
    ),
)


def kernel(y_pred, s):
    y2 = y_pred.reshape(_ROWS, _COLS)
    s2 = s.reshape(_ROWS, _COLS)
    out = _tc_reduce(y2, s2)
    return out[0, 0]
